# baseline (device time: 109312 ns/iter reference)
import jax
import jax.numpy as jnp
from jax import lax
from jax.experimental import pallas as pl
from jax.experimental.pallas import tpu as pltpu

N_DEV = 4
B_SH = 64
D = 2048
H_SH = 4096
B = N_DEV * B_SH
KT = 2048
N_T = H_SH // KT


def kernel(x, Win0, Wout0, Win1, Wout1, Win2, Wout2):
    def body(x_ref, win0, wout0, win1, wout1, win2, wout2, out_ref,
             xg, h_buf, partial, comm_ag, comm_rs, wstage,
             ag_s, ag_r, rs_s, rs_r, dma_sems):
        my = lax.axis_index("i")
        left = (my - 1) % N_DEV
        right = (my + 1) % N_DEV
        diag = (my + 2) % N_DEV

        barrier = pltpu.get_barrier_semaphore()
        for nbr in (left, right, diag):
            pl.semaphore_signal(barrier, inc=1, device_id=(nbr,),
                                device_id_type=pl.DeviceIdType.MESH)
        pl.semaphore_wait(barrier, 3)

        wins = [win0, win1, win2]
        wouts = [wout0, wout1, wout2]

        def issue_w(l, t, slot):
            if t % 2 == 0:
                src = wins[l].at[:, pl.ds((t // 2) * KT, KT)]
            else:
                src = wouts[l].at[pl.ds((t // 2) * KT, KT), :]
            cp = pltpu.make_async_copy(src, wstage.at[slot], dma_sems.at[slot])
            cp.start()
            return cp

        def wtile(slot):
            return wstage[slot].astype(jnp.bfloat16)

        def pchunk(c):
            return partial[pl.ds(c * B_SH, B_SH), :]

        comm_ag[0] = x_ref[...].astype(jnp.bfloat16)

        cp0 = issue_w(0, 0, 0)
        cp1 = issue_w(0, 1, 1)

        for l in range(3):
            rR = pltpu.make_async_remote_copy(
                src_ref=comm_ag.at[0], dst_ref=comm_ag.at[1],
                send_sem=ag_s.at[0], recv_sem=ag_r.at[0],
                device_id=(right,), device_id_type=pl.DeviceIdType.MESH)
            rL = pltpu.make_async_remote_copy(
                src_ref=comm_ag.at[0], dst_ref=comm_ag.at[2],
                send_sem=ag_s.at[1], recv_sem=ag_r.at[1],
                device_id=(left,), device_id_type=pl.DeviceIdType.MESH)
            rD = pltpu.make_async_remote_copy(
                src_ref=comm_ag.at[0], dst_ref=comm_ag.at[3],
                send_sem=ag_s.at[2], recv_sem=ag_r.at[2],
                device_id=(diag,), device_id_type=pl.DeviceIdType.MESH)
            rR.start()
            rL.start()
            rD.start()
            xg[pl.ds(my * B_SH, B_SH), :] = comm_ag[0]
            rR.wait()
            xg[pl.ds(left * B_SH, B_SH), :] = comm_ag[1]
            rL.wait()
            xg[pl.ds(right * B_SH, B_SH), :] = comm_ag[2]
            rD.wait()
            xg[pl.ds(diag * B_SH, B_SH), :] = comm_ag[3]

            xg_v = xg[...]
            cp0.wait()
            h0 = jnp.maximum(
                jnp.dot(xg_v, wtile(0), preferred_element_type=jnp.float32),
                0.0).astype(jnp.bfloat16)
            cp2 = issue_w(l, 2, 0)
            cp1.wait()
            partial[...] = jnp.dot(h0, wtile(1),
                                   preferred_element_type=jnp.float32)
            cp3 = issue_w(l, 3, 1)
            cp2.wait()
            h_buf[...] = jnp.maximum(
                jnp.dot(xg_v, wtile(0), preferred_element_type=jnp.float32),
                0.0).astype(jnp.bfloat16)
            if l < 2:
                cp0 = issue_w(l + 1, 0, 0)
            cp3.wait()
            w_o1 = wtile(1)
            if l < 2:
                cp1 = issue_w(l + 1, 1, 1)

            def pfin(c):
                hc = h_buf[pl.ds(c * B_SH, B_SH), :]
                partial[pl.ds(c * B_SH, B_SH), :] = (
                    pchunk(c) + jnp.dot(hc, w_o1,
                                        preferred_element_type=jnp.float32))

            pfin(right)
            comm_rs[0] = pchunk(right).astype(jnp.bfloat16)
            sR = pltpu.make_async_remote_copy(
                src_ref=comm_rs.at[0], dst_ref=comm_rs.at[3],
                send_sem=rs_s.at[0], recv_sem=rs_r.at[0],
                device_id=(right,), device_id_type=pl.DeviceIdType.MESH)
            sR.start()
            pfin(left)
            comm_rs[1] = pchunk(left).astype(jnp.bfloat16)
            sL = pltpu.make_async_remote_copy(
                src_ref=comm_rs.at[1], dst_ref=comm_rs.at[4],
                send_sem=rs_s.at[1], recv_sem=rs_r.at[1],
                device_id=(left,), device_id_type=pl.DeviceIdType.MESH)
            sL.start()
            pfin(diag)
            comm_rs[2] = pchunk(diag).astype(jnp.bfloat16)
            sD = pltpu.make_async_remote_copy(
                src_ref=comm_rs.at[2], dst_ref=comm_rs.at[5],
                send_sem=rs_s.at[2], recv_sem=rs_r.at[2],
                device_id=(diag,), device_id_type=pl.DeviceIdType.MESH)
            sD.start()
            pfin(my)
            sR.wait()
            sL.wait()
            sD.wait()
            result = (pchunk(my) + comm_rs[3].astype(jnp.float32)
                      + comm_rs[4].astype(jnp.float32)
                      + comm_rs[5].astype(jnp.float32))

            if l < 2:
                comm_ag[0] = result.astype(jnp.bfloat16)
            else:
                out_ref[...] = result

    return pl.pallas_call(
        body,
        out_shape=jax.ShapeDtypeStruct((B_SH, D), jnp.float32),
        in_specs=[
            pl.BlockSpec(memory_space=pltpu.MemorySpace.VMEM),
            pl.BlockSpec(memory_space=pltpu.MemorySpace.HBM),
            pl.BlockSpec(memory_space=pltpu.MemorySpace.HBM),
            pl.BlockSpec(memory_space=pltpu.MemorySpace.HBM),
            pl.BlockSpec(memory_space=pltpu.MemorySpace.HBM),
            pl.BlockSpec(memory_space=pltpu.MemorySpace.HBM),
            pl.BlockSpec(memory_space=pltpu.MemorySpace.HBM),
        ],
        out_specs=pl.BlockSpec(memory_space=pltpu.MemorySpace.VMEM),
        scratch_shapes=[
            pltpu.VMEM((B, D), jnp.bfloat16),
            pltpu.VMEM((B, KT), jnp.bfloat16),
            pltpu.VMEM((B, D), jnp.float32),
            pltpu.VMEM((N_DEV, B_SH, D), jnp.bfloat16),
            pltpu.VMEM((6, B_SH, D), jnp.bfloat16),
            pltpu.VMEM((2, D, KT), jnp.float32),
            pltpu.SemaphoreType.DMA((3,)),
            pltpu.SemaphoreType.DMA((3,)),
            pltpu.SemaphoreType.DMA((3,)),
            pltpu.SemaphoreType.DMA((3,)),
            pltpu.SemaphoreType.DMA((2,)),
        ],
        compiler_params=pltpu.CompilerParams(
            collective_id=0, vmem_limit_bytes=60 * 1024 * 1024),
    )(x, Win0, Wout0, Win1, Wout1, Win2, Wout2)


# device time: 106650 ns/iter; 1.0250x vs baseline; 1.0250x over previous
import jax
import jax.numpy as jnp
from jax import lax
from jax.experimental import pallas as pl
from jax.experimental.pallas import tpu as pltpu

N_DEV = 4
B_SH = 64
D = 2048
H_SH = 4096
B = N_DEV * B_SH
KT = 2048
N_T = H_SH // KT


def kernel(x, Win0, Wout0, Win1, Wout1, Win2, Wout2):
    def body(x_ref, win0, wout0, win1, wout1, win2, wout2, out_ref,
             xg, h_buf, partial, comm_ag, comm_rs, wstage,
             ag_s, ag_r, rs_s, rs_r, dma_sems):
        my = lax.axis_index("i")
        left = (my - 1) % N_DEV
        right = (my + 1) % N_DEV
        diag = (my + 2) % N_DEV

        barrier = pltpu.get_barrier_semaphore()
        for nbr in (left, right, diag):
            pl.semaphore_signal(barrier, inc=1, device_id=(nbr,),
                                device_id_type=pl.DeviceIdType.MESH)
        pl.semaphore_wait(barrier, 3)

        wins = [win0, win1, win2]
        wouts = [wout0, wout1, wout2]

        def issue_w(l, t, slot):
            if t % 2 == 0:
                src = wins[l].at[:, pl.ds((t // 2) * KT, KT)]
            else:
                src = wouts[l].at[pl.ds((t // 2) * KT, KT), :]
            cp = pltpu.make_async_copy(src, wstage.at[slot], dma_sems.at[slot])
            cp.start()
            return cp

        def wtile(slot):
            return wstage[slot].astype(jnp.bfloat16)

        def pchunk(c):
            return partial[pl.ds(c * B_SH, B_SH), :]

        comm_ag[0] = x_ref[...].astype(jnp.bfloat16)

        cp0 = issue_w(0, 0, 0)
        cp1 = issue_w(0, 1, 1)

        for l in range(3):
            rR = pltpu.make_async_remote_copy(
                src_ref=comm_ag.at[0], dst_ref=comm_ag.at[1],
                send_sem=ag_s.at[0], recv_sem=ag_r.at[0],
                device_id=(right,), device_id_type=pl.DeviceIdType.MESH)
            rL = pltpu.make_async_remote_copy(
                src_ref=comm_ag.at[0], dst_ref=comm_ag.at[2],
                send_sem=ag_s.at[1], recv_sem=ag_r.at[1],
                device_id=(left,), device_id_type=pl.DeviceIdType.MESH)
            rD = pltpu.make_async_remote_copy(
                src_ref=comm_ag.at[0], dst_ref=comm_ag.at[3],
                send_sem=ag_s.at[2], recv_sem=ag_r.at[2],
                device_id=(diag,), device_id_type=pl.DeviceIdType.MESH)
            rR.start()
            rL.start()
            rD.start()
            xg[pl.ds(my * B_SH, B_SH), :] = comm_ag[0]
            rR.wait()
            xg[pl.ds(left * B_SH, B_SH), :] = comm_ag[1]
            rL.wait()
            xg[pl.ds(right * B_SH, B_SH), :] = comm_ag[2]
            rD.wait()
            xg[pl.ds(diag * B_SH, B_SH), :] = comm_ag[3]

            xg_v = xg[...]
            cp0.wait()
            h0 = jnp.maximum(
                jnp.dot(xg_v, wtile(0), preferred_element_type=jnp.float32),
                0.0).astype(jnp.bfloat16)
            cp2 = issue_w(l, 2, 0)
            cp1.wait()
            partial[...] = jnp.dot(h0, wtile(1),
                                   preferred_element_type=jnp.float32)
            cp3 = issue_w(l, 3, 1)
            cp2.wait()
            h_buf[...] = jnp.maximum(
                jnp.dot(xg_v, wtile(0), preferred_element_type=jnp.float32),
                0.0).astype(jnp.bfloat16)
            cp3.wait()
            w_o1 = wtile(1)

            if l < 2:
                cp0 = issue_w(l + 1, 0, 0)
                cp1 = issue_w(l + 1, 1, 1)

            def pfin(c):
                hc = h_buf[pl.ds(c * B_SH, B_SH), :]
                partial[pl.ds(c * B_SH, B_SH), :] = (
                    pchunk(c) + jnp.dot(hc, w_o1,
                                        preferred_element_type=jnp.float32))

            pfin(right)
            comm_rs[0] = pchunk(right).astype(jnp.bfloat16)
            sR = pltpu.make_async_remote_copy(
                src_ref=comm_rs.at[0], dst_ref=comm_rs.at[3],
                send_sem=rs_s.at[0], recv_sem=rs_r.at[0],
                device_id=(right,), device_id_type=pl.DeviceIdType.MESH)
            sR.start()
            pfin(left)
            comm_rs[1] = pchunk(left).astype(jnp.bfloat16)
            sL = pltpu.make_async_remote_copy(
                src_ref=comm_rs.at[1], dst_ref=comm_rs.at[4],
                send_sem=rs_s.at[1], recv_sem=rs_r.at[1],
                device_id=(left,), device_id_type=pl.DeviceIdType.MESH)
            sL.start()
            pfin(diag)
            comm_rs[2] = pchunk(diag).astype(jnp.bfloat16)
            sD = pltpu.make_async_remote_copy(
                src_ref=comm_rs.at[2], dst_ref=comm_rs.at[5],
                send_sem=rs_s.at[2], recv_sem=rs_r.at[2],
                device_id=(diag,), device_id_type=pl.DeviceIdType.MESH)
            sD.start()
            pfin(my)
            sR.wait()
            sL.wait()
            sD.wait()
            result = (pchunk(my) + comm_rs[3].astype(jnp.float32)
                      + comm_rs[4].astype(jnp.float32)
                      + comm_rs[5].astype(jnp.float32))

            if l < 2:
                comm_ag[0] = result.astype(jnp.bfloat16)
            else:
                out_ref[...] = result

    return pl.pallas_call(
        body,
        out_shape=jax.ShapeDtypeStruct((B_SH, D), jnp.float32),
        in_specs=[
            pl.BlockSpec(memory_space=pltpu.MemorySpace.VMEM),
            pl.BlockSpec(memory_space=pltpu.MemorySpace.HBM),
            pl.BlockSpec(memory_space=pltpu.MemorySpace.HBM),
            pl.BlockSpec(memory_space=pltpu.MemorySpace.HBM),
            pl.BlockSpec(memory_space=pltpu.MemorySpace.HBM),
            pl.BlockSpec(memory_space=pltpu.MemorySpace.HBM),
            pl.BlockSpec(memory_space=pltpu.MemorySpace.HBM),
        ],
        out_specs=pl.BlockSpec(memory_space=pltpu.MemorySpace.VMEM),
        scratch_shapes=[
            pltpu.VMEM((B, D), jnp.bfloat16),
            pltpu.VMEM((B, KT), jnp.bfloat16),
            pltpu.VMEM((B, D), jnp.float32),
            pltpu.VMEM((N_DEV, B_SH, D), jnp.bfloat16),
            pltpu.VMEM((6, B_SH, D), jnp.bfloat16),
            pltpu.VMEM((2, D, KT), jnp.float32),
            pltpu.SemaphoreType.DMA((3,)),
            pltpu.SemaphoreType.DMA((3,)),
            pltpu.SemaphoreType.DMA((3,)),
            pltpu.SemaphoreType.DMA((3,)),
            pltpu.SemaphoreType.DMA((2,)),
        ],
        compiler_params=pltpu.CompilerParams(
            collective_id=0, vmem_limit_bytes=60 * 1024 * 1024),
    )(x, Win0, Wout0, Win1, Wout1, Win2, Wout2)
